# baseline (speedup 1.0000x reference)
import jax
import jax.numpy as jnp
from jax import lax
from jax.experimental import pallas as pl
from jax.experimental.pallas import tpu as pltpu

E = 64
TOP_K = 2
DIM = 768
HID = 2048
T = 32
P = T * TOP_K


def _gate_kernel(x_ref, gw_ref, sidx_ref, dn_ref, wt_ref):
    xf = x_ref[...]
    gw = gw_ref[...]
    logits = jax.lax.dot_general(xf, gw, (((1,), (1,)), ((), ())),
                                 preferred_element_type=jnp.float32)
    m = jnp.max(logits, axis=1, keepdims=True)
    p = jnp.exp(logits - m)
    prob = p / jnp.sum(p, axis=1, keepdims=True)

    cols = jax.lax.broadcasted_iota(jnp.int32, (T, E), 1)
    m1 = jnp.max(prob, axis=1, keepdims=True)
    i1 = jnp.min(jnp.where(prob == m1, cols, E), axis=1, keepdims=True)
    pm = jnp.where(cols == i1, -1.0, prob)
    m2 = jnp.max(pm, axis=1, keepdims=True)
    i2 = jnp.min(jnp.where(pm == m2, cols, E), axis=1, keepdims=True)
    s = m1 + m2 + 1e-20
    w1n = m1 / s
    w2n = m2 / s

    wt = (jnp.where(cols == i1, w1n, 0.0)
          + jnp.where(cols == i2, w2n, 0.0))
    wt_ref[...] = wt

    used = jnp.max(jnp.where(wt > 0.0, 1, 0), axis=0, keepdims=True)
    dn_ref[...] = jnp.sum(used, axis=1, keepdims=True)

    e_mat = jnp.concatenate([i1, i2], axis=1)
    qid = (jax.lax.broadcasted_iota(jnp.int32, (T, TOP_K), 0)
           + T * jax.lax.broadcasted_iota(jnp.int32, (T, TOP_K), 1))
    key0 = e_mat * P + qid
    pcols = jax.lax.broadcasted_iota(jnp.int32, (1, P), 1)
    big = jnp.int32(E * P + P)

    def body(i, carry):
        key, se = carry
        mk = jnp.min(key)
        e = mk // P
        se = jnp.where(pcols == i, jnp.minimum(e, E - 1), se)
        key = jnp.where(key // P == e, big, key)
        return key, se

    _, se = lax.fori_loop(0, P, body, (key0, jnp.zeros((1, P), jnp.int32)))
    sidx_ref[...] = se


def _ffn_kernel(sidx_ref, dn_ref, x_ref, wt_ref, w1_hbm, w3_hbm, w2_hbm,
                out_ref, w1b, w3b, w2b, sems):
    num = dn_ref[0, 0]

    def copies(i, slot):
        e = sidx_ref[0, i]
        h2 = HID // 2
        d2 = DIM // 2
        return (
            pltpu.make_async_copy(w1_hbm.at[e, pl.ds(0, h2)],
                                  w1b.at[slot, pl.ds(0, h2)], sems.at[slot, 0]),
            pltpu.make_async_copy(w1_hbm.at[e, pl.ds(h2, h2)],
                                  w1b.at[slot, pl.ds(h2, h2)], sems.at[slot, 1]),
            pltpu.make_async_copy(w3_hbm.at[e, pl.ds(0, h2)],
                                  w3b.at[slot, pl.ds(0, h2)], sems.at[slot, 2]),
            pltpu.make_async_copy(w3_hbm.at[e, pl.ds(h2, h2)],
                                  w3b.at[slot, pl.ds(h2, h2)], sems.at[slot, 3]),
            pltpu.make_async_copy(w2_hbm.at[e, pl.ds(0, d2)],
                                  w2b.at[slot, pl.ds(0, d2)], sems.at[slot, 4]),
            pltpu.make_async_copy(w2_hbm.at[e, pl.ds(d2, d2)],
                                  w2b.at[slot, pl.ds(d2, d2)], sems.at[slot, 5]),
        )

    for c in copies(0, 0):
        c.start()
    out_ref[...] = jnp.zeros_like(out_ref)
    xf = x_ref[...]
    cols = jax.lax.broadcasted_iota(jnp.int32, (T, E), 1)
    wt = wt_ref[...]

    def body(i, carry):
        slot = lax.rem(i, 2)

        @pl.when(i + 1 < num)
        def _prefetch():
            for c in copies(i + 1, 1 - slot):
                c.start()

        for c in copies(i, slot):
            c.wait()

        w1v = w1b[pl.ds(slot, 1)][0]
        w3v = w3b[pl.ds(slot, 1)][0]
        w2v = w2b[pl.ds(slot, 1)][0]
        a = jax.lax.dot_general(xf, w1v, (((1,), (1,)), ((), ())),
                                preferred_element_type=jnp.float32)
        b = jax.lax.dot_general(xf, w3v, (((1,), (1,)), ((), ())),
                                preferred_element_type=jnp.float32)
        h = a * jax.nn.sigmoid(a) * b
        o = jax.lax.dot_general(h, w2v, (((1,), (1,)), ((), ())),
                                preferred_element_type=jnp.float32)
        e = sidx_ref[0, i]
        wcol = jnp.sum(jnp.where(cols == e, wt, 0.0),
                       axis=1, keepdims=True)
        out_ref[...] = out_ref[...] + o * wcol
        return carry

    lax.fori_loop(0, num, body, 0)


def kernel(x, gate_w, w1, w2, w3):
    orig_shape = x.shape
    xf = x.reshape(-1, DIM)

    sidx, dn, wt = pl.pallas_call(
        _gate_kernel,
        out_shape=(
            jax.ShapeDtypeStruct((1, P), jnp.int32),
            jax.ShapeDtypeStruct((1, 1), jnp.int32),
            jax.ShapeDtypeStruct((T, E), jnp.float32),
        ),
    )(xf, gate_w)

    out = pl.pallas_call(
        _ffn_kernel,
        in_specs=[
            pl.BlockSpec(memory_space=pltpu.SMEM),
            pl.BlockSpec(memory_space=pltpu.SMEM),
            pl.BlockSpec(memory_space=pltpu.VMEM),
            pl.BlockSpec(memory_space=pltpu.VMEM),
            pl.BlockSpec(memory_space=pl.ANY),
            pl.BlockSpec(memory_space=pl.ANY),
            pl.BlockSpec(memory_space=pl.ANY),
        ],
        out_shape=jax.ShapeDtypeStruct((T, DIM), jnp.float32),
        scratch_shapes=[
            pltpu.VMEM((2, HID, DIM), jnp.float32),
            pltpu.VMEM((2, HID, DIM), jnp.float32),
            pltpu.VMEM((2, DIM, HID), jnp.float32),
            pltpu.SemaphoreType.DMA((2, 6)),
        ],
    )(sidx, dn, xf, wt, w1, w3, w2)

    return out.reshape(orig_shape)


# speedup vs baseline: 1.0443x; 1.0443x over previous
import jax
import jax.numpy as jnp
from jax import lax
from jax.experimental import pallas as pl
from jax.experimental.pallas import tpu as pltpu

E = 64
TOP_K = 2
DIM = 768
HID = 2048
T = 32
P = T * TOP_K
PW = P + 8


def _moe_kernel(x_ref, gw_ref, w1_hbm, w3_hbm, w2_hbm, out_ref,
                w1b, w3b, w2b, sems, disp_v, disp_s, dsem):
    xf = x_ref[...]
    gw = gw_ref[...]
    logits = jax.lax.dot_general(xf, gw, (((1,), (1,)), ((), ())),
                                 preferred_element_type=jnp.float32)
    m = jnp.max(logits, axis=1, keepdims=True)
    p = jnp.exp(logits - m)
    prob = p / jnp.sum(p, axis=1, keepdims=True)

    cols = jax.lax.broadcasted_iota(jnp.int32, (T, E), 1)
    m1 = jnp.max(prob, axis=1, keepdims=True)
    i1 = jnp.min(jnp.where(prob == m1, cols, E), axis=1, keepdims=True)
    pm = jnp.where(cols == i1, -1.0, prob)
    m2 = jnp.max(pm, axis=1, keepdims=True)
    i2 = jnp.min(jnp.where(pm == m2, cols, E), axis=1, keepdims=True)
    s = m1 + m2 + 1e-20
    w1n = m1 / s
    w2n = m2 / s

    wt = (jnp.where(cols == i1, w1n, 0.0)
          + jnp.where(cols == i2, w2n, 0.0))

    used = jnp.max(jnp.where(wt > 0.0, 1, 0), axis=0, keepdims=True)
    dnum = jnp.sum(used, axis=1, keepdims=True)

    e_mat = jnp.concatenate([i1, i2], axis=1)
    qid = (jax.lax.broadcasted_iota(jnp.int32, (T, TOP_K), 0)
           + T * jax.lax.broadcasted_iota(jnp.int32, (T, TOP_K), 1))
    key0 = e_mat * P + qid
    pcols = jax.lax.broadcasted_iota(jnp.int32, (1, PW), 1)
    big = jnp.int32(E * P + P)

    def sbody(i, carry):
        key, se = carry
        mk = jnp.min(key)
        e = mk // P
        se = jnp.where(pcols == i, jnp.minimum(e, E - 1), se)
        key = jnp.where(key // P == e, big, key)
        return key, se

    _, se = lax.fori_loop(0, P, sbody,
                          (key0, jnp.zeros((1, PW), jnp.int32)))
    se = jnp.where(pcols == P, dnum, se)

    disp_v[...] = se
    dcopy = pltpu.make_async_copy(disp_v, disp_s, dsem)
    dcopy.start()
    dcopy.wait()
    num = disp_s[0, P]

    def copies(i, slot):
        e = disp_s[0, i]
        return (
            pltpu.make_async_copy(w1_hbm.at[e], w1b.at[slot], sems.at[slot, 0]),
            pltpu.make_async_copy(w3_hbm.at[e], w3b.at[slot], sems.at[slot, 1]),
            pltpu.make_async_copy(w2_hbm.at[e], w2b.at[slot], sems.at[slot, 2]),
        )

    for c in copies(0, 0):
        c.start()
    out_ref[...] = jnp.zeros_like(out_ref)

    def body(i, carry):
        slot = lax.rem(i, 2)

        @pl.when(i + 1 < num)
        def _prefetch():
            for c in copies(i + 1, 1 - slot):
                c.start()

        for c in copies(i, slot):
            c.wait()

        w1v = w1b[pl.ds(slot, 1)][0]
        w3v = w3b[pl.ds(slot, 1)][0]
        w2v = w2b[pl.ds(slot, 1)][0]
        a = jax.lax.dot_general(xf, w1v, (((1,), (1,)), ((), ())),
                                preferred_element_type=jnp.float32)
        b = jax.lax.dot_general(xf, w3v, (((1,), (1,)), ((), ())),
                                preferred_element_type=jnp.float32)
        h = a * jax.nn.sigmoid(a) * b
        o = jax.lax.dot_general(h, w2v, (((1,), (1,)), ((), ())),
                                preferred_element_type=jnp.float32)
        e = disp_s[0, i]
        wcol = jnp.sum(jnp.where(cols == e, wt, 0.0),
                       axis=1, keepdims=True)
        out_ref[...] = out_ref[...] + o * wcol
        return carry

    lax.fori_loop(0, num, body, 0)


def kernel(x, gate_w, w1, w2, w3):
    orig_shape = x.shape
    xf = x.reshape(-1, DIM)

    out = pl.pallas_call(
        _moe_kernel,
        in_specs=[
            pl.BlockSpec(memory_space=pltpu.VMEM),
            pl.BlockSpec(memory_space=pltpu.VMEM),
            pl.BlockSpec(memory_space=pl.ANY),
            pl.BlockSpec(memory_space=pl.ANY),
            pl.BlockSpec(memory_space=pl.ANY),
        ],
        out_shape=jax.ShapeDtypeStruct((T, DIM), jnp.float32),
        scratch_shapes=[
            pltpu.VMEM((2, HID, DIM), jnp.float32),
            pltpu.VMEM((2, HID, DIM), jnp.float32),
            pltpu.VMEM((2, DIM, HID), jnp.float32),
            pltpu.SemaphoreType.DMA((2, 3)),
            pltpu.VMEM((1, PW), jnp.int32),
            pltpu.SMEM((1, PW), jnp.int32),
            pltpu.SemaphoreType.DMA,
        ],
    )(xf, gate_w, w1, w3, w2)

    return out.reshape(orig_shape)


# speedup vs baseline: 1.0611x; 1.0161x over previous
import jax
import jax.numpy as jnp
from jax import lax
from jax.experimental import pallas as pl
from jax.experimental.pallas import tpu as pltpu

E = 64
TOP_K = 2
DIM = 768
HID = 2048
T = 32
P = T * TOP_K
PW = P + 8


def _moe_kernel(x_ref, gw_ref, w1_hbm, w3_hbm, w2_hbm, out_ref,
                w1b, w3b, w2b, sems, disp_v, disp_s, dsem):
    xf = x_ref[...]
    gw = gw_ref[...]
    logits = jax.lax.dot_general(xf, gw, (((1,), (1,)), ((), ())),
                                 preferred_element_type=jnp.float32)
    m = jnp.max(logits, axis=1, keepdims=True)
    p = jnp.exp(logits - m)
    prob = p / jnp.sum(p, axis=1, keepdims=True)

    cols = jax.lax.broadcasted_iota(jnp.int32, (T, E), 1)
    m1 = jnp.max(prob, axis=1, keepdims=True)
    i1 = jnp.min(jnp.where(prob == m1, cols, E), axis=1, keepdims=True)
    pm = jnp.where(cols == i1, -1.0, prob)
    m2 = jnp.max(pm, axis=1, keepdims=True)
    i2 = jnp.min(jnp.where(pm == m2, cols, E), axis=1, keepdims=True)
    s = m1 + m2 + 1e-20
    w1n = m1 / s
    w2n = m2 / s

    wt = (jnp.where(cols == i1, w1n, 0.0)
          + jnp.where(cols == i2, w2n, 0.0))

    used = jnp.max(jnp.where(wt > 0.0, 1, 0), axis=0, keepdims=True)
    dnum = jnp.sum(used, axis=1, keepdims=True)

    e_mat = jnp.concatenate([i1, i2], axis=1)
    qid = (jax.lax.broadcasted_iota(jnp.int32, (T, TOP_K), 0)
           + T * jax.lax.broadcasted_iota(jnp.int32, (T, TOP_K), 1))
    key0 = e_mat * P + qid
    pcols = jax.lax.broadcasted_iota(jnp.int32, (1, PW), 1)
    big = jnp.int32(E * P + P)

    disp_v[...] = jnp.broadcast_to(jnp.min(e_mat, axis=0, keepdims=True)
                                   .min(axis=1, keepdims=True), (1, PW))
    dcopy0 = pltpu.make_async_copy(disp_v, disp_s, dsem)
    dcopy0.start()
    dcopy0.wait()
    e0 = disp_s[0, 0]
    c1f = pltpu.make_async_copy(w1_hbm.at[e0], w1b.at[0], sems.at[0, 0])
    c3f = pltpu.make_async_copy(w3_hbm.at[e0], w3b.at[0], sems.at[0, 1])
    c2f = pltpu.make_async_copy(w2_hbm.at[e0], w2b.at[0], sems.at[0, 2])
    c1f.start()
    c3f.start()
    c2f.start()

    def sbody(i, carry):
        key, se = carry
        mk = jnp.min(key)
        e = mk // P
        se = jnp.where(pcols == i, jnp.minimum(e, E - 1), se)
        key = jnp.where(key // P == e, big, key)
        return key, se

    _, se = lax.fori_loop(0, P, sbody,
                          (key0, jnp.zeros((1, PW), jnp.int32)))
    se = jnp.where(pcols == P, dnum, se)

    disp_v[...] = se
    dcopy = pltpu.make_async_copy(disp_v, disp_s, dsem)
    dcopy.start()
    dcopy.wait()
    num = disp_s[0, P]

    def copies(i, slot):
        e = disp_s[0, i]
        return (
            pltpu.make_async_copy(w1_hbm.at[e], w1b.at[slot], sems.at[slot, 0]),
            pltpu.make_async_copy(w3_hbm.at[e], w3b.at[slot], sems.at[slot, 1]),
            pltpu.make_async_copy(w2_hbm.at[e], w2b.at[slot], sems.at[slot, 2]),
        )

    out_ref[...] = jnp.zeros_like(out_ref)

    def body(i, carry):
        slot = lax.rem(i, 2)

        @pl.when(i + 1 < num)
        def _prefetch():
            for c in copies(i + 1, 1 - slot):
                c.start()

        c1, c3, c2 = copies(i, slot)
        c1.wait()
        w1v = w1b[pl.ds(slot, 1)][0]
        a = jax.lax.dot_general(xf, w1v, (((1,), (1,)), ((), ())),
                                preferred_element_type=jnp.float32)
        c3.wait()
        w3v = w3b[pl.ds(slot, 1)][0]
        b = jax.lax.dot_general(xf, w3v, (((1,), (1,)), ((), ())),
                                preferred_element_type=jnp.float32)
        h = a * jax.nn.sigmoid(a) * b
        c2.wait()
        w2v = w2b[pl.ds(slot, 1)][0]
        o = jax.lax.dot_general(h, w2v, (((1,), (1,)), ((), ())),
                                preferred_element_type=jnp.float32)
        e = disp_s[0, i]
        wcol = jnp.sum(jnp.where(cols == e, wt, 0.0),
                       axis=1, keepdims=True)
        out_ref[...] = out_ref[...] + o * wcol
        return carry

    lax.fori_loop(0, num, body, 0)


def kernel(x, gate_w, w1, w2, w3):
    orig_shape = x.shape
    xf = x.reshape(-1, DIM)

    out = pl.pallas_call(
        _moe_kernel,
        in_specs=[
            pl.BlockSpec(memory_space=pltpu.VMEM),
            pl.BlockSpec(memory_space=pltpu.VMEM),
            pl.BlockSpec(memory_space=pl.ANY),
            pl.BlockSpec(memory_space=pl.ANY),
            pl.BlockSpec(memory_space=pl.ANY),
        ],
        out_shape=jax.ShapeDtypeStruct((T, DIM), jnp.float32),
        scratch_shapes=[
            pltpu.VMEM((2, HID, DIM), jnp.float32),
            pltpu.VMEM((2, HID, DIM), jnp.float32),
            pltpu.VMEM((2, DIM, HID), jnp.float32),
            pltpu.SemaphoreType.DMA((2, 3)),
            pltpu.VMEM((1, PW), jnp.int32),
            pltpu.SMEM((1, PW), jnp.int32),
            pltpu.SemaphoreType.DMA,
        ],
    )(xf, gate_w, w1, w3, w2)

    return out.reshape(orig_shape)


# speedup vs baseline: 1.0613x; 1.0002x over previous
import jax
import jax.numpy as jnp
from jax import lax
from jax.experimental import pallas as pl
from jax.experimental.pallas import tpu as pltpu

E = 64
TOP_K = 2
DIM = 768
HID = 2048
T = 32
P = T * TOP_K
PW = P + 8


def _moe_kernel(x_ref, gw_ref, w1_hbm, w3_hbm, w2_hbm, out_ref,
                w1b, w3b, w2b, sems, disp_v, disp_s, dsem):
    xf = x_ref[...]
    gw = gw_ref[...]
    logits = jax.lax.dot_general(xf, gw, (((1,), (1,)), ((), ())),
                                 preferred_element_type=jnp.float32)
    m = jnp.max(logits, axis=1, keepdims=True)
    p = jnp.exp(logits - m)
    prob = p / jnp.sum(p, axis=1, keepdims=True)

    cols = jax.lax.broadcasted_iota(jnp.int32, (T, E), 1)
    m1 = jnp.max(prob, axis=1, keepdims=True)
    i1 = jnp.min(jnp.where(prob == m1, cols, E), axis=1, keepdims=True)
    pm = jnp.where(cols == i1, -1.0, prob)
    m2 = jnp.max(pm, axis=1, keepdims=True)
    i2 = jnp.min(jnp.where(pm == m2, cols, E), axis=1, keepdims=True)
    s = m1 + m2 + 1e-20
    w1n = m1 / s
    w2n = m2 / s

    wt = (jnp.where(cols == i1, w1n, 0.0)
          + jnp.where(cols == i2, w2n, 0.0))

    used = jnp.max(jnp.where(wt > 0.0, 1, 0), axis=0, keepdims=True)
    dnum = jnp.sum(used, axis=1, keepdims=True)

    e_mat = jnp.concatenate([i1, i2], axis=1)
    w_mat = jnp.concatenate([w1n, w2n], axis=1)
    qid = (jax.lax.broadcasted_iota(jnp.int32, (T, TOP_K), 0)
           + T * jax.lax.broadcasted_iota(jnp.int32, (T, TOP_K), 1))
    pcols = jax.lax.broadcasted_iota(jnp.int32, (1, PW), 1)
    big = jnp.int32(E * P + P)
    key0 = jnp.where(w_mat > 0.0, e_mat * P + qid, big)

    emin = jnp.min(jnp.where(w_mat > 0.0, e_mat, E),
                   axis=0, keepdims=True).min(axis=1, keepdims=True)
    disp_v[...] = jnp.broadcast_to(emin, (1, PW))
    dcopy0 = pltpu.make_async_copy(disp_v, disp_s, dsem)
    dcopy0.start()
    dcopy0.wait()
    e0 = disp_s[0, 0]
    c1f = pltpu.make_async_copy(w1_hbm.at[e0], w1b.at[0], sems.at[0, 0])
    c3f = pltpu.make_async_copy(w3_hbm.at[e0], w3b.at[0], sems.at[0, 1])
    c2f = pltpu.make_async_copy(w2_hbm.at[e0], w2b.at[0], sems.at[0, 2])
    c1f.start()
    c3f.start()
    c2f.start()

    def sbody(i, carry):
        key, se = carry
        mk = jnp.min(key)
        e = mk // P
        se = jnp.where(pcols == i, jnp.minimum(e, E - 1), se)
        key = jnp.where(key // P == e, big, key)
        return key, se

    _, se = lax.fori_loop(0, P, sbody,
                          (key0, jnp.zeros((1, PW), jnp.int32)))
    se = jnp.where(pcols == P, dnum, se)

    disp_v[...] = se
    dcopy = pltpu.make_async_copy(disp_v, disp_s, dsem)
    dcopy.start()
    dcopy.wait()
    num = disp_s[0, P]

    def copies(i, slot):
        e = disp_s[0, i]
        return (
            pltpu.make_async_copy(w1_hbm.at[e], w1b.at[slot], sems.at[slot, 0]),
            pltpu.make_async_copy(w3_hbm.at[e], w3b.at[slot], sems.at[slot, 1]),
            pltpu.make_async_copy(w2_hbm.at[e], w2b.at[slot], sems.at[slot, 2]),
        )

    out_ref[...] = jnp.zeros_like(out_ref)

    def body(i, carry):
        slot = lax.rem(i, 2)

        @pl.when(i + 1 < num)
        def _prefetch():
            for c in copies(i + 1, 1 - slot):
                c.start()

        c1, c3, c2 = copies(i, slot)
        c1.wait()
        w1v = w1b[pl.ds(slot, 1)][0]
        a = jax.lax.dot_general(xf, w1v, (((1,), (1,)), ((), ())),
                                preferred_element_type=jnp.float32)
        c3.wait()
        w3v = w3b[pl.ds(slot, 1)][0]
        b = jax.lax.dot_general(xf, w3v, (((1,), (1,)), ((), ())),
                                preferred_element_type=jnp.float32)
        h = a * jax.nn.sigmoid(a) * b
        c2.wait()
        w2v = w2b[pl.ds(slot, 1)][0]
        o = jax.lax.dot_general(h, w2v, (((1,), (1,)), ((), ())),
                                preferred_element_type=jnp.float32)
        e = disp_s[0, i]
        wcol = jnp.sum(jnp.where(cols == e, wt, 0.0),
                       axis=1, keepdims=True)
        out_ref[...] = out_ref[...] + o * wcol
        return carry

    lax.fori_loop(0, num, body, 0)


def kernel(x, gate_w, w1, w2, w3):
    orig_shape = x.shape
    xf = x.reshape(-1, DIM)

    out = pl.pallas_call(
        _moe_kernel,
        in_specs=[
            pl.BlockSpec(memory_space=pltpu.VMEM),
            pl.BlockSpec(memory_space=pltpu.VMEM),
            pl.BlockSpec(memory_space=pl.ANY),
            pl.BlockSpec(memory_space=pl.ANY),
            pl.BlockSpec(memory_space=pl.ANY),
        ],
        out_shape=jax.ShapeDtypeStruct((T, DIM), jnp.float32),
        scratch_shapes=[
            pltpu.VMEM((2, HID, DIM), jnp.float32),
            pltpu.VMEM((2, HID, DIM), jnp.float32),
            pltpu.VMEM((2, DIM, HID), jnp.float32),
            pltpu.SemaphoreType.DMA((2, 3)),
            pltpu.VMEM((1, PW), jnp.int32),
            pltpu.SMEM((1, PW), jnp.int32),
            pltpu.SemaphoreType.DMA,
        ],
    )(xf, gate_w, w1, w3, w2)

    return out.reshape(orig_shape)
